# baseline (speedup 1.0000x reference)
import functools

import jax
import jax.numpy as jnp
from jax import lax
from jax.experimental import pallas as pl
from jax.experimental.pallas import tpu as pltpu
from jax.experimental.pallas import tpu_sc as plsc

NC = 2
NS = 16
NW = NC * NS
CH = 80
NBUF = 6
LAG = 2

_MESH = plsc.VectorSubcoreMesh(core_axis_name="c", subcore_axis_name="s")
_SC_PARAMS = pltpu.CompilerParams(use_tc_tiling_on_sc=False)


def _row_part(n):
    rpt = ((n // NS + 7) // 8) * 8
    tail = n - rpt * (NS - 1)
    assert 0 < tail <= rpt and tail % 8 == 0
    return rpt, tail


def _per_tile_rows(s, n, copy_fn):
    rpt, tail = _row_part(n)

    @pl.when(s < NS - 1)
    def _():
        copy_fn(s * rpt, rpt)

    @pl.when(s == NS - 1)
    def _():
        copy_fn((NS - 1) * rpt, tail)


def _deg_body(nchunks, n, dst_hbm, ones_hbm, zeros_hbm, out_hbm,
              dst_v, ones_v, deg_sh, sem):
    c = lax.axis_index("c")
    s = lax.axis_index("s")

    def init(r0, nr):
        @pl.when(c == 0)
        def _():
            pltpu.sync_copy(ones_hbm.at[pl.ds(0, nr), :],
                            deg_sh.at[pl.ds(r0, nr), :])

        @pl.when(c == 1)
        def _():
            pltpu.sync_copy(zeros_hbm.at[pl.ds(0, nr), :],
                            deg_sh.at[pl.ds(r0, nr), :])

    _per_tile_rows(s, n, init)
    pltpu.sync_copy(ones_hbm.at[pl.ds(0, CH), :], ones_v)
    pltpu.sync_copy(dst_hbm.at[s, pl.ds(c * nchunks, nchunks), :], dst_v)
    plsc.subcore_barrier()

    def step(j, carry):
        pltpu.async_copy(ones_v, deg_sh.at[dst_v.at[j]], sem, add=True)

        @pl.when(j >= 8)
        def _():
            pltpu.make_async_copy(ones_v, deg_sh.at[dst_v.at[j]], sem).wait()

        return carry

    lax.fori_loop(0, nchunks, step, 0)

    def drain(j, carry):
        pltpu.make_async_copy(ones_v, deg_sh.at[dst_v.at[0]], sem).wait()
        return carry

    lax.fori_loop(0, 8, drain, 0)
    plsc.subcore_barrier()

    def writeout(r0, nr):
        pltpu.sync_copy(deg_sh.at[pl.ds(r0, nr), :],
                        out_hbm.at[c, pl.ds(r0, nr), :])

    _per_tile_rows(s, n, writeout)


def _agg_body(nchunks, n, ys_hbm, src_hbm, dst_hbm, out_hbm,
              src_v, dst_v, rows_v, acc_sh, sem_g, sem_s):
    c = lax.axis_index("c")
    s = lax.axis_index("s")
    yc = ys_hbm.at[c]

    def init(r0, nr):
        pltpu.sync_copy(yc.at[pl.ds(r0, nr), :], acc_sh.at[pl.ds(r0, nr), :])

    _per_tile_rows(s, n, init)
    pltpu.sync_copy(src_hbm.at[s], src_v)
    pltpu.sync_copy(dst_hbm.at[s], dst_v)
    plsc.subcore_barrier()

    for b in range(NBUF - LAG):
        pltpu.async_copy(yc.at[src_v.at[b]], rows_v.at[b], sem_g)

    def step(j, carry):
        b = lax.rem(j, NBUF)
        pltpu.make_async_copy(yc.at[src_v.at[j]], rows_v.at[b], sem_g).wait()
        pltpu.async_copy(rows_v.at[b], acc_sh.at[dst_v.at[j]], sem_s,
                         add=True)

        @pl.when(j >= LAG)
        def _():
            pltpu.make_async_copy(rows_v.at[0], acc_sh.at[dst_v.at[0]],
                                  sem_s).wait()

        @pl.when(j + NBUF - LAG < nchunks)
        def _():
            pltpu.async_copy(yc.at[src_v.at[j + NBUF - LAG]],
                             rows_v.at[lax.rem(j + NBUF - LAG, NBUF)], sem_g)

        return carry

    lax.fori_loop(0, nchunks, step, 0)

    def drain(j, carry):
        pltpu.make_async_copy(rows_v.at[0], acc_sh.at[dst_v.at[0]],
                              sem_s).wait()
        return carry

    lax.fori_loop(0, LAG, drain, 0)
    plsc.subcore_barrier()

    def writeout(r0, nr):
        pltpu.sync_copy(acc_sh.at[pl.ds(r0, nr), :],
                        out_hbm.at[c, pl.ds(r0, nr), :])

    _per_tile_rows(s, n, writeout)


def _tc1a_body(x_ref, w1_ref, y_ref):
    y_ref[...] = jnp.dot(x_ref[...], w1_ref[...],
                         preferred_element_type=jnp.float32)


def _tc1b_body(y_ref, degp_ref, dinv_ref, ys_ref):
    p = degp_ref[0] + degp_ref[1]
    dinv = lax.rsqrt(p[:, 0:1])
    dinv_ref[...] = dinv
    ys = y_ref[...] * dinv
    dh = ys.shape[1] // 2
    ys_ref[0] = ys[:, :dh]
    ys_ref[1] = ys[:, dh:]


def _bn(z, g, be):
    m = jnp.mean(z, axis=0, keepdims=True)
    zc = z - m
    v = jnp.mean(zc * zc, axis=0, keepdims=True)
    return zc * lax.rsqrt(v + 1e-5) * g + be


def _tc2_body(acc_ref, dinv_ref, b1_ref, g1_ref, be1_ref, w2_ref, ys_ref):
    dinv = dinv_ref[...]
    z = jnp.concatenate([acc_ref[0], acc_ref[1]], axis=1) * dinv + b1_ref[...]
    h = _bn(z, g1_ref[...], be1_ref[...])
    h = jnp.where(h > 0, h, 0.01 * h)
    y = jnp.dot(h, w2_ref[...], preferred_element_type=jnp.float32)
    ys = y * dinv
    dh = ys.shape[1] // 2
    ys_ref[0] = ys[:, :dh]
    ys_ref[1] = ys[:, dh:]


def _tc3_body(acc_ref, dinv_ref, b2_ref, g2_ref, be2_ref, x_ref, out_ref):
    z = (jnp.concatenate([acc_ref[0], acc_ref[1]], axis=1) * dinv_ref[...]
         + b2_ref[...])
    h = _bn(z, g2_ref[...], be2_ref[...])
    t = h + x_ref[...]
    out_ref[...] = jnp.where(t > 0, t, 0.01 * t)


@functools.lru_cache(maxsize=None)
def _build(n, e, d):
    assert e % NW == 0 and (e // NS) % CH == 0 and n % 8 == 0 and d % 2 == 0
    epw = e // NW
    eps = e // NS
    dh = d // 2

    deg_call = pl.kernel(
        functools.partial(_deg_body, epw // CH, n),
        out_type=jax.ShapeDtypeStruct((NC, n, 16), jnp.float32),
        mesh=_MESH,
        scratch_types=[
            pltpu.VMEM((epw // CH, CH), jnp.int32),
            pltpu.VMEM((CH, 16), jnp.float32),
            pltpu.VMEM_SHARED((n, 16), jnp.float32),
            pltpu.SemaphoreType.DMA,
        ],
        compiler_params=_SC_PARAMS,
    )

    agg_call = pl.kernel(
        functools.partial(_agg_body, eps // CH, n),
        out_type=jax.ShapeDtypeStruct((NC, n, dh), jnp.float32),
        mesh=_MESH,
        scratch_types=[
            pltpu.VMEM((eps // CH, CH), jnp.int32),
            pltpu.VMEM((eps // CH, CH), jnp.int32),
            pltpu.VMEM((NBUF, CH, dh), jnp.float32),
            pltpu.VMEM_SHARED((n, dh), jnp.float32),
            pltpu.SemaphoreType.DMA,
            pltpu.SemaphoreType.DMA,
        ],
        compiler_params=_SC_PARAMS,
    )

    tc1a = pl.pallas_call(
        _tc1a_body,
        out_shape=jax.ShapeDtypeStruct((n, d), jnp.float32),
    )
    tc1b = pl.pallas_call(
        _tc1b_body,
        out_shape=(jax.ShapeDtypeStruct((n, 1), jnp.float32),
                   jax.ShapeDtypeStruct((NC, n, dh), jnp.float32)),
    )
    tc2 = pl.pallas_call(
        _tc2_body,
        out_shape=jax.ShapeDtypeStruct((NC, n, dh), jnp.float32),
    )
    tc3 = pl.pallas_call(
        _tc3_body,
        out_shape=jax.ShapeDtypeStruct((n, d), jnp.float32),
    )
    return deg_call, agg_call, tc1a, tc1b, tc2, tc3


def kernel(x, edge_index, W1, b1, g1, be1, W2, b2, g2, be2):
    n, d = x.shape
    e = edge_index.shape[1]
    deg_call, agg_call, tc1a, tc1b, tc2, tc3 = _build(n, e, d)

    dst = edge_index[1]
    dst_s = dst.reshape(NS, (e // NS) // CH, CH)
    src = lax.optimization_barrier(edge_index)[0]
    src_s = src.reshape(NS, (e // NS) // CH, CH)
    rpt, _ = _row_part(n)
    ones16 = jnp.ones((rpt, 16), jnp.float32)
    zeros16 = jnp.zeros((rpt, 16), jnp.float32)
    b1r, g1r, be1r = b1.reshape(1, d), g1.reshape(1, d), be1.reshape(1, d)
    b2r, g2r, be2r = b2.reshape(1, d), g2.reshape(1, d), be2.reshape(1, d)

    degp = deg_call(dst_s, ones16, zeros16)
    y1 = tc1a(x, W1)
    dinv, ys = tc1b(y1, degp)
    acc1 = agg_call(ys, src_s, dst_s)
    ys2 = tc2(acc1, dinv, b1r, g1r, be1r, W2)
    acc2 = agg_call(ys2, src_s, dst_s)
    out = tc3(acc2, dinv, b2r, g2r, be2r, x)
    return out


# speedup vs baseline: 1.0188x; 1.0188x over previous
import functools

import jax
import jax.numpy as jnp
from jax import lax
from jax.experimental import pallas as pl
from jax.experimental.pallas import tpu as pltpu
from jax.experimental.pallas import tpu_sc as plsc

NC = 2
NS = 16
NW = NC * NS
CH = 80
NBUF = 6
LAG = 2

_MESH = plsc.VectorSubcoreMesh(core_axis_name="c", subcore_axis_name="s")
_SC_PARAMS = pltpu.CompilerParams(use_tc_tiling_on_sc=False)


def _row_part(n):
    rpt = ((n // NS + 7) // 8) * 8
    tail = n - rpt * (NS - 1)
    assert 0 < tail <= rpt and tail % 8 == 0
    return rpt, tail


def _per_tile_rows(s, n, copy_fn):
    rpt, tail = _row_part(n)

    @pl.when(s < NS - 1)
    def _():
        copy_fn(s * rpt, rpt)

    @pl.when(s == NS - 1)
    def _():
        copy_fn((NS - 1) * rpt, tail)


def _deg_body(nchunks, n, dst_hbm, ones_hbm, zeros_hbm, out_hbm,
              dst_v, ones_v, deg_sh, sem):
    c = lax.axis_index("c")
    s = lax.axis_index("s")

    def init(r0, nr):
        @pl.when(c == 0)
        def _():
            pltpu.sync_copy(ones_hbm.at[pl.ds(0, nr), :],
                            deg_sh.at[pl.ds(r0, nr), :])

        @pl.when(c == 1)
        def _():
            pltpu.sync_copy(zeros_hbm.at[pl.ds(0, nr), :],
                            deg_sh.at[pl.ds(r0, nr), :])

    _per_tile_rows(s, n, init)
    pltpu.sync_copy(ones_hbm.at[pl.ds(0, CH), :], ones_v)
    pltpu.sync_copy(dst_hbm.at[s, pl.ds(c * nchunks, nchunks), :], dst_v)
    plsc.subcore_barrier()

    def step(j, carry):
        pltpu.async_copy(ones_v, deg_sh.at[dst_v.at[j]], sem, add=True)

        @pl.when(j >= 8)
        def _():
            pltpu.make_async_copy(ones_v, deg_sh.at[dst_v.at[j]], sem).wait()

        return carry

    lax.fori_loop(0, nchunks, step, 0)

    def drain(j, carry):
        pltpu.make_async_copy(ones_v, deg_sh.at[dst_v.at[0]], sem).wait()
        return carry

    lax.fori_loop(0, 8, drain, 0)
    plsc.subcore_barrier()

    def writeout(r0, nr):
        pltpu.sync_copy(deg_sh.at[pl.ds(r0, nr), :],
                        out_hbm.at[c, pl.ds(r0, nr), :])

    _per_tile_rows(s, n, writeout)


def _agg_body(nchunks, n, ys_hbm, src_hbm, dst_hbm, out_hbm,
              src_v, dst_v, rows_v, acc_sh, sem_g, sem_s):
    c = lax.axis_index("c")
    s = lax.axis_index("s")
    yc = ys_hbm.at[c]

    def init(r0, nr):
        pltpu.sync_copy(yc.at[pl.ds(r0, nr), :], acc_sh.at[pl.ds(r0, nr), :])

    _per_tile_rows(s, n, init)
    pltpu.sync_copy(src_hbm.at[s], src_v)
    pltpu.sync_copy(dst_hbm.at[s], dst_v)
    plsc.subcore_barrier()

    for b in range(NBUF - LAG):
        pltpu.async_copy(yc.at[src_v.at[b]], rows_v.at[b], sem_g)

    def step(j, carry):
        b = lax.rem(j, NBUF)
        pltpu.make_async_copy(yc.at[src_v.at[j]], rows_v.at[b], sem_g).wait()
        pltpu.async_copy(rows_v.at[b], acc_sh.at[dst_v.at[j]], sem_s,
                         add=True)

        @pl.when(j >= LAG)
        def _():
            pltpu.make_async_copy(rows_v.at[0], acc_sh.at[dst_v.at[0]],
                                  sem_s).wait()

        @pl.when(j + NBUF - LAG < nchunks)
        def _():
            pltpu.async_copy(yc.at[src_v.at[j + NBUF - LAG]],
                             rows_v.at[lax.rem(j + NBUF - LAG, NBUF)], sem_g)

        return carry

    lax.fori_loop(0, nchunks, step, 0)

    def drain(j, carry):
        pltpu.make_async_copy(rows_v.at[0], acc_sh.at[dst_v.at[0]],
                              sem_s).wait()
        return carry

    lax.fori_loop(0, LAG, drain, 0)
    plsc.subcore_barrier()

    def writeout(r0, nr):
        pltpu.sync_copy(acc_sh.at[pl.ds(r0, nr), :],
                        out_hbm.at[c, pl.ds(r0, nr), :])

    _per_tile_rows(s, n, writeout)


def _tc1_body(x_ref, w1_ref, degp_ref, dinv_ref, ys_ref):
    p = degp_ref[0] + degp_ref[1]
    dinv = lax.rsqrt(p[:, 0:1])
    dinv_ref[...] = dinv
    y = jnp.dot(x_ref[...], w1_ref[...], preferred_element_type=jnp.float32)
    ys = y * dinv
    dh = ys.shape[1] // 2
    ys_ref[0] = ys[:, :dh]
    ys_ref[1] = ys[:, dh:]


def _bn(z, g, be):
    m = jnp.mean(z, axis=0, keepdims=True)
    zc = z - m
    v = jnp.mean(zc * zc, axis=0, keepdims=True)
    return zc * lax.rsqrt(v + 1e-5) * g + be


def _tc2_body(acc_ref, dinv_ref, b1_ref, g1_ref, be1_ref, w2_ref, ys_ref):
    dinv = dinv_ref[...]
    z = jnp.concatenate([acc_ref[0], acc_ref[1]], axis=1) * dinv + b1_ref[...]
    h = _bn(z, g1_ref[...], be1_ref[...])
    h = jnp.where(h > 0, h, 0.01 * h)
    y = jnp.dot(h, w2_ref[...], preferred_element_type=jnp.float32)
    ys = y * dinv
    dh = ys.shape[1] // 2
    ys_ref[0] = ys[:, :dh]
    ys_ref[1] = ys[:, dh:]


def _tc3_body(acc_ref, dinv_ref, b2_ref, g2_ref, be2_ref, x_ref, out_ref):
    z = (jnp.concatenate([acc_ref[0], acc_ref[1]], axis=1) * dinv_ref[...]
         + b2_ref[...])
    h = _bn(z, g2_ref[...], be2_ref[...])
    t = h + x_ref[...]
    out_ref[...] = jnp.where(t > 0, t, 0.01 * t)


@functools.lru_cache(maxsize=None)
def _build(n, e, d):
    assert e % NW == 0 and (e // NS) % CH == 0 and n % 8 == 0 and d % 2 == 0
    epw = e // NW
    eps = e // NS
    dh = d // 2

    deg_call = pl.kernel(
        functools.partial(_deg_body, epw // CH, n),
        out_type=jax.ShapeDtypeStruct((NC, n, 16), jnp.float32),
        mesh=_MESH,
        scratch_types=[
            pltpu.VMEM((epw // CH, CH), jnp.int32),
            pltpu.VMEM((CH, 16), jnp.float32),
            pltpu.VMEM_SHARED((n, 16), jnp.float32),
            pltpu.SemaphoreType.DMA,
        ],
        compiler_params=_SC_PARAMS,
    )

    agg_call = pl.kernel(
        functools.partial(_agg_body, eps // CH, n),
        out_type=jax.ShapeDtypeStruct((NC, n, dh), jnp.float32),
        mesh=_MESH,
        scratch_types=[
            pltpu.VMEM((eps // CH, CH), jnp.int32),
            pltpu.VMEM((eps // CH, CH), jnp.int32),
            pltpu.VMEM((NBUF, CH, dh), jnp.float32),
            pltpu.VMEM_SHARED((n, dh), jnp.float32),
            pltpu.SemaphoreType.DMA,
            pltpu.SemaphoreType.DMA,
        ],
        compiler_params=_SC_PARAMS,
    )

    tc1 = pl.pallas_call(
        _tc1_body,
        out_shape=(jax.ShapeDtypeStruct((n, 1), jnp.float32),
                   jax.ShapeDtypeStruct((NC, n, dh), jnp.float32)),
    )
    tc2 = pl.pallas_call(
        _tc2_body,
        out_shape=jax.ShapeDtypeStruct((NC, n, dh), jnp.float32),
    )
    tc3 = pl.pallas_call(
        _tc3_body,
        out_shape=jax.ShapeDtypeStruct((n, d), jnp.float32),
    )
    return deg_call, agg_call, tc1, tc2, tc3


def kernel(x, edge_index, W1, b1, g1, be1, W2, b2, g2, be2):
    n, d = x.shape
    e = edge_index.shape[1]
    deg_call, agg_call, tc1, tc2, tc3 = _build(n, e, d)

    src = edge_index[0]
    dst = edge_index[1]
    src_s = src.reshape(NS, (e // NS) // CH, CH)
    dst_s = dst.reshape(NS, (e // NS) // CH, CH)
    rpt, _ = _row_part(n)
    ones16 = jnp.ones((rpt, 16), jnp.float32)
    zeros16 = jnp.zeros((rpt, 16), jnp.float32)
    b1r, g1r, be1r = b1.reshape(1, d), g1.reshape(1, d), be1.reshape(1, d)
    b2r, g2r, be2r = b2.reshape(1, d), g2.reshape(1, d), be2.reshape(1, d)

    degp = deg_call(dst_s, ones16, zeros16)
    dinv, ys = tc1(x, W1, degp)
    acc1 = agg_call(ys, src_s, dst_s)
    ys2 = tc2(acc1, dinv, b1r, g1r, be1r, W2)
    acc2 = agg_call(ys2, src_s, dst_s)
    out = tc3(acc2, dinv, b2r, g2r, be2r, x)
    return out


# speedup vs baseline: 1.0672x; 1.0475x over previous
import functools

import jax
import jax.numpy as jnp
from jax import lax
from jax.experimental import pallas as pl
from jax.experimental.pallas import tpu as pltpu
from jax.experimental.pallas import tpu_sc as plsc

NC = 2
NS = 16
NW = NC * NS
CH = 80
NBUF = 6
LAG = 2

_MESH = plsc.VectorSubcoreMesh(core_axis_name="c", subcore_axis_name="s")
_SC_PARAMS = pltpu.CompilerParams(use_tc_tiling_on_sc=False)


def _row_part(n):
    rpt = ((n // NS + 7) // 8) * 8
    tail = n - rpt * (NS - 1)
    assert 0 < tail <= rpt and tail % 8 == 0
    return rpt, tail


def _per_tile_rows(s, n, copy_fn):
    rpt, tail = _row_part(n)

    @pl.when(s < NS - 1)
    def _():
        copy_fn(s * rpt, rpt)

    @pl.when(s == NS - 1)
    def _():
        copy_fn((NS - 1) * rpt, tail)


def _deg_body(nchunks, n, dst_hbm, ones_hbm, zeros_hbm, out_hbm,
              dst_v, ones_v, deg_sh, sem):
    c = lax.axis_index("c")
    s = lax.axis_index("s")

    def init(r0, nr):
        @pl.when(c == 0)
        def _():
            pltpu.sync_copy(ones_hbm.at[pl.ds(0, nr)],
                            deg_sh.at[pl.ds(r0, nr)])

        @pl.when(c == 1)
        def _():
            pltpu.sync_copy(zeros_hbm.at[pl.ds(0, nr)],
                            deg_sh.at[pl.ds(r0, nr)])

    _per_tile_rows(s, n, init)
    pltpu.sync_copy(ones_hbm.at[pl.ds(0, CH)], ones_v)
    pltpu.sync_copy(dst_hbm.at[s, pl.ds(c * nchunks, nchunks), :], dst_v)
    plsc.subcore_barrier()

    def step(j, carry):
        pltpu.async_copy(ones_v, deg_sh.at[dst_v.at[j]], sem, add=True)

        @pl.when(j >= 8)
        def _():
            pltpu.make_async_copy(ones_v, deg_sh.at[dst_v.at[j]], sem).wait()

        return carry

    lax.fori_loop(0, nchunks, step, 0)

    def drain(j, carry):
        pltpu.make_async_copy(ones_v, deg_sh.at[dst_v.at[0]], sem).wait()
        return carry

    lax.fori_loop(0, 8, drain, 0)
    plsc.subcore_barrier()

    def writeout(r0, nr):
        pltpu.sync_copy(deg_sh.at[pl.ds(r0, nr)],
                        out_hbm.at[c, pl.ds(r0, nr)])

    _per_tile_rows(s, n, writeout)


def _agg_body(nchunks, n, ys_hbm, src_hbm, dst_hbm, out_hbm,
              src_v, dst_v, rows_v, acc_sh, sem_g, sem_s):
    c = lax.axis_index("c")
    s = lax.axis_index("s")
    yc = ys_hbm.at[c]

    def init(r0, nr):
        pltpu.sync_copy(yc.at[pl.ds(r0, nr), :], acc_sh.at[pl.ds(r0, nr), :])

    _per_tile_rows(s, n, init)
    pltpu.sync_copy(src_hbm.at[s], src_v)
    pltpu.sync_copy(dst_hbm.at[s], dst_v)
    plsc.subcore_barrier()

    for b in range(NBUF - LAG):
        pltpu.async_copy(yc.at[src_v.at[b]], rows_v.at[b], sem_g)

    def step(j, carry):
        b = lax.rem(j, NBUF)
        pltpu.make_async_copy(yc.at[src_v.at[j]], rows_v.at[b], sem_g).wait()
        pltpu.async_copy(rows_v.at[b], acc_sh.at[dst_v.at[j]], sem_s,
                         add=True)

        @pl.when(j >= LAG)
        def _():
            pltpu.make_async_copy(rows_v.at[0], acc_sh.at[dst_v.at[0]],
                                  sem_s).wait()

        @pl.when(j + NBUF - LAG < nchunks)
        def _():
            pltpu.async_copy(yc.at[src_v.at[j + NBUF - LAG]],
                             rows_v.at[lax.rem(j + NBUF - LAG, NBUF)], sem_g)

        return carry

    lax.fori_loop(0, nchunks, step, 0)

    def drain(j, carry):
        pltpu.make_async_copy(rows_v.at[0], acc_sh.at[dst_v.at[0]],
                              sem_s).wait()
        return carry

    lax.fori_loop(0, LAG, drain, 0)
    plsc.subcore_barrier()

    def writeout(r0, nr):
        pltpu.sync_copy(acc_sh.at[pl.ds(r0, nr), :],
                        out_hbm.at[c, pl.ds(r0, nr), :])

    _per_tile_rows(s, n, writeout)


def _tc1_body(x_ref, w1_ref, degp_ref, dinv_ref, ys_ref):
    p = degp_ref[0] + degp_ref[1]
    dinv = lax.rsqrt(p)[:, None]
    dinv_ref[...] = dinv
    y = jnp.dot(x_ref[...], w1_ref[...], preferred_element_type=jnp.float32)
    ys = y * dinv
    dh = ys.shape[1] // 2
    ys_ref[0] = ys[:, :dh]
    ys_ref[1] = ys[:, dh:]


def _bn(z, g, be):
    m = jnp.mean(z, axis=0, keepdims=True)
    zc = z - m
    v = jnp.mean(zc * zc, axis=0, keepdims=True)
    return zc * lax.rsqrt(v + 1e-5) * g + be


def _tc2_body(acc_ref, dinv_ref, b1_ref, g1_ref, be1_ref, w2_ref, ys_ref):
    dinv = dinv_ref[...]
    z = jnp.concatenate([acc_ref[0], acc_ref[1]], axis=1) * dinv + b1_ref[...]
    h = _bn(z, g1_ref[...], be1_ref[...])
    h = jnp.where(h > 0, h, 0.01 * h)
    y = jnp.dot(h, w2_ref[...], preferred_element_type=jnp.float32)
    ys = y * dinv
    dh = ys.shape[1] // 2
    ys_ref[0] = ys[:, :dh]
    ys_ref[1] = ys[:, dh:]


def _tc3_body(acc_ref, dinv_ref, b2_ref, g2_ref, be2_ref, x_ref, out_ref):
    z = (jnp.concatenate([acc_ref[0], acc_ref[1]], axis=1) * dinv_ref[...]
         + b2_ref[...])
    h = _bn(z, g2_ref[...], be2_ref[...])
    t = h + x_ref[...]
    out_ref[...] = jnp.where(t > 0, t, 0.01 * t)


@functools.lru_cache(maxsize=None)
def _build(n, e, d):
    assert e % NW == 0 and (e // NS) % CH == 0 and n % 8 == 0 and d % 2 == 0
    epw = e // NW
    eps = e // NS
    dh = d // 2

    deg_call = pl.kernel(
        functools.partial(_deg_body, epw // CH, n),
        out_type=jax.ShapeDtypeStruct((NC, n), jnp.float32),
        mesh=_MESH,
        scratch_types=[
            pltpu.VMEM((epw // CH, CH), jnp.int32),
            pltpu.VMEM((CH,), jnp.float32),
            pltpu.VMEM_SHARED((n,), jnp.float32),
            pltpu.SemaphoreType.DMA,
        ],
        compiler_params=_SC_PARAMS,
    )

    agg_call = pl.kernel(
        functools.partial(_agg_body, eps // CH, n),
        out_type=jax.ShapeDtypeStruct((NC, n, dh), jnp.float32),
        mesh=_MESH,
        scratch_types=[
            pltpu.VMEM((eps // CH, CH), jnp.int32),
            pltpu.VMEM((eps // CH, CH), jnp.int32),
            pltpu.VMEM((NBUF, CH, dh), jnp.float32),
            pltpu.VMEM_SHARED((n, dh), jnp.float32),
            pltpu.SemaphoreType.DMA,
            pltpu.SemaphoreType.DMA,
        ],
        compiler_params=_SC_PARAMS,
    )

    tc1 = pl.pallas_call(
        _tc1_body,
        out_shape=(jax.ShapeDtypeStruct((n, 1), jnp.float32),
                   jax.ShapeDtypeStruct((NC, n, dh), jnp.float32)),
    )
    tc2 = pl.pallas_call(
        _tc2_body,
        out_shape=jax.ShapeDtypeStruct((NC, n, dh), jnp.float32),
    )
    tc3 = pl.pallas_call(
        _tc3_body,
        out_shape=jax.ShapeDtypeStruct((n, d), jnp.float32),
    )
    return deg_call, agg_call, tc1, tc2, tc3


def kernel(x, edge_index, W1, b1, g1, be1, W2, b2, g2, be2):
    n, d = x.shape
    e = edge_index.shape[1]
    deg_call, agg_call, tc1, tc2, tc3 = _build(n, e, d)

    src = edge_index[0]
    dst = edge_index[1]
    src_s = src.reshape(NS, (e // NS) // CH, CH)
    dst_s = dst.reshape(NS, (e // NS) // CH, CH)
    rpt, _ = _row_part(n)
    ones16 = jnp.ones((rpt,), jnp.float32)
    zeros16 = jnp.zeros((rpt,), jnp.float32)
    b1r, g1r, be1r = b1.reshape(1, d), g1.reshape(1, d), be1.reshape(1, d)
    b2r, g2r, be2r = b2.reshape(1, d), g2.reshape(1, d), be2.reshape(1, d)

    degp = deg_call(dst_s, ones16, zeros16)
    dinv, ys = tc1(x, W1, degp)
    acc1 = agg_call(ys, src_s, dst_s)
    ys2 = tc2(acc1, dinv, b1r, g1r, be1r, W2)
    acc2 = agg_call(ys2, src_s, dst_s)
    out = tc3(acc2, dinv, b2r, g2r, be2r, x)
    return out


# speedup vs baseline: 1.0743x; 1.0067x over previous
import functools

import jax
import jax.numpy as jnp
from jax import lax
from jax.experimental import pallas as pl
from jax.experimental.pallas import tpu as pltpu
from jax.experimental.pallas import tpu_sc as plsc

NC = 2
NS = 16
NW = NC * NS
CH = 80
NBUF = 6
LAG = 2

_MESH = plsc.VectorSubcoreMesh(core_axis_name="c", subcore_axis_name="s")
_SC_PARAMS = pltpu.CompilerParams(use_tc_tiling_on_sc=False)


def _row_part(n):
    rpt = ((n // NS + 7) // 8) * 8
    tail = n - rpt * (NS - 1)
    assert 0 < tail <= rpt and tail % 8 == 0
    return rpt, tail


def _per_tile_rows(s, n, copy_fn):
    rpt, tail = _row_part(n)

    @pl.when(s < NS - 1)
    def _():
        copy_fn(s * rpt, rpt)

    @pl.when(s == NS - 1)
    def _():
        copy_fn((NS - 1) * rpt, tail)


def _deg_body(nchunks, n, dst_hbm, ones_hbm, zeros_hbm, out_hbm,
              dst_v, ones_v, deg_sh, sem):
    c = lax.axis_index("c")
    s = lax.axis_index("s")

    def init(r0, nr):
        @pl.when(c == 0)
        def _():
            pltpu.sync_copy(ones_hbm.at[pl.ds(0, nr)],
                            deg_sh.at[pl.ds(r0, nr)])

        @pl.when(c == 1)
        def _():
            pltpu.sync_copy(zeros_hbm.at[pl.ds(0, nr)],
                            deg_sh.at[pl.ds(r0, nr)])

    _per_tile_rows(s, n, init)
    pltpu.sync_copy(ones_hbm.at[pl.ds(0, CH)], ones_v)
    pltpu.sync_copy(dst_hbm.at[s, pl.ds(c * nchunks, nchunks), :], dst_v)
    plsc.subcore_barrier()

    def step(j, carry):
        pltpu.async_copy(ones_v, deg_sh.at[dst_v.at[j]], sem, add=True)

        @pl.when(j >= 8)
        def _():
            pltpu.make_async_copy(ones_v, deg_sh.at[dst_v.at[j]], sem).wait()

        return carry

    lax.fori_loop(0, nchunks, step, 0)

    def drain(j, carry):
        pltpu.make_async_copy(ones_v, deg_sh.at[dst_v.at[0]], sem).wait()
        return carry

    lax.fori_loop(0, 8, drain, 0)
    plsc.subcore_barrier()

    def writeout(r0, nr):
        pltpu.sync_copy(deg_sh.at[pl.ds(r0, nr)],
                        out_hbm.at[c, pl.ds(r0, nr)])

    _per_tile_rows(s, n, writeout)


def _agg_body(nchunks, n, ys_hbm, src_hbm, dst_hbm, out_hbm,
              src_v, dst_v, rows_v, acc_sh, sem_g, sem_s):
    c = lax.axis_index("c")
    s = lax.axis_index("s")
    yc = ys_hbm.at[c]

    def init(r0, nr):
        pltpu.sync_copy(yc.at[pl.ds(r0, nr), :], acc_sh.at[pl.ds(r0, nr), :])

    _per_tile_rows(s, n, init)
    pltpu.sync_copy(src_hbm.at[s], src_v)
    pltpu.sync_copy(dst_hbm.at[s], dst_v)
    plsc.subcore_barrier()

    for b in range(NBUF - LAG):
        pltpu.async_copy(yc.at[src_v.at[b]], rows_v.at[b], sem_g)

    def step(j, carry):
        b = lax.rem(j, NBUF)
        pltpu.make_async_copy(yc.at[src_v.at[j]], rows_v.at[b], sem_g).wait()
        pltpu.async_copy(rows_v.at[b], acc_sh.at[dst_v.at[j]], sem_s,
                         add=True)

        @pl.when(j >= LAG)
        def _():
            pltpu.make_async_copy(rows_v.at[0], acc_sh.at[dst_v.at[0]],
                                  sem_s).wait()

        @pl.when(j + NBUF - LAG < nchunks)
        def _():
            pltpu.async_copy(yc.at[src_v.at[j + NBUF - LAG]],
                             rows_v.at[lax.rem(j + NBUF - LAG, NBUF)], sem_g)

        return carry

    lax.fori_loop(0, nchunks, step, 0)

    def drain(j, carry):
        pltpu.make_async_copy(rows_v.at[0], acc_sh.at[dst_v.at[0]],
                              sem_s).wait()
        return carry

    lax.fori_loop(0, LAG, drain, 0)
    plsc.subcore_barrier()

    def writeout(r0, nr):
        pltpu.sync_copy(acc_sh.at[pl.ds(r0, nr), :],
                        out_hbm.at[c, pl.ds(r0, nr), :])

    _per_tile_rows(s, n, writeout)


def _dinv(degp_ref):
    p = degp_ref[0] + degp_ref[1]
    return lax.rsqrt(p)[:, None]


def _tc1_body(x_ref, w1_ref, degp_ref, ys_ref):
    y = jnp.dot(x_ref[...], w1_ref[...], preferred_element_type=jnp.float32)
    ys = y * _dinv(degp_ref)
    dh = ys.shape[1] // 2
    ys_ref[0] = ys[:, :dh]
    ys_ref[1] = ys[:, dh:]


def _bn(z, g, be):
    m = jnp.mean(z, axis=0, keepdims=True)
    zc = z - m
    v = jnp.mean(zc * zc, axis=0, keepdims=True)
    return zc * lax.rsqrt(v + 1e-5) * g + be


def _tc2_body(acc_ref, degp_ref, b1_ref, g1_ref, be1_ref, w2_ref, ys_ref):
    dinv = _dinv(degp_ref)
    z = jnp.concatenate([acc_ref[0], acc_ref[1]], axis=1) * dinv + b1_ref[...]
    h = _bn(z, g1_ref[...], be1_ref[...])
    h = jnp.where(h > 0, h, 0.01 * h)
    y = jnp.dot(h, w2_ref[...], preferred_element_type=jnp.float32)
    ys = y * dinv
    dh = ys.shape[1] // 2
    ys_ref[0] = ys[:, :dh]
    ys_ref[1] = ys[:, dh:]


def _tc3_body(acc_ref, degp_ref, b2_ref, g2_ref, be2_ref, x_ref, out_ref):
    z = (jnp.concatenate([acc_ref[0], acc_ref[1]], axis=1) * _dinv(degp_ref)
         + b2_ref[...])
    h = _bn(z, g2_ref[...], be2_ref[...])
    t = h + x_ref[...]
    out_ref[...] = jnp.where(t > 0, t, 0.01 * t)


@functools.lru_cache(maxsize=None)
def _build(n, e, d):
    assert e % NW == 0 and (e // NS) % CH == 0 and n % 8 == 0 and d % 2 == 0
    epw = e // NW
    eps = e // NS
    dh = d // 2

    deg_call = pl.kernel(
        functools.partial(_deg_body, epw // CH, n),
        out_type=jax.ShapeDtypeStruct((NC, n), jnp.float32),
        mesh=_MESH,
        scratch_types=[
            pltpu.VMEM((epw // CH, CH), jnp.int32),
            pltpu.VMEM((CH,), jnp.float32),
            pltpu.VMEM_SHARED((n,), jnp.float32),
            pltpu.SemaphoreType.DMA,
        ],
        compiler_params=_SC_PARAMS,
    )

    agg_call = pl.kernel(
        functools.partial(_agg_body, eps // CH, n),
        out_type=jax.ShapeDtypeStruct((NC, n, dh), jnp.float32),
        mesh=_MESH,
        scratch_types=[
            pltpu.VMEM((eps // CH, CH), jnp.int32),
            pltpu.VMEM((eps // CH, CH), jnp.int32),
            pltpu.VMEM((NBUF, CH, dh), jnp.float32),
            pltpu.VMEM_SHARED((n, dh), jnp.float32),
            pltpu.SemaphoreType.DMA,
            pltpu.SemaphoreType.DMA,
        ],
        compiler_params=_SC_PARAMS,
    )

    tc1 = pl.pallas_call(
        _tc1_body,
        out_shape=jax.ShapeDtypeStruct((NC, n, dh), jnp.float32),
    )
    tc2 = pl.pallas_call(
        _tc2_body,
        out_shape=jax.ShapeDtypeStruct((NC, n, dh), jnp.float32),
    )
    tc3 = pl.pallas_call(
        _tc3_body,
        out_shape=jax.ShapeDtypeStruct((n, d), jnp.float32),
    )
    return deg_call, agg_call, tc1, tc2, tc3


def kernel(x, edge_index, W1, b1, g1, be1, W2, b2, g2, be2):
    n, d = x.shape
    e = edge_index.shape[1]
    deg_call, agg_call, tc1, tc2, tc3 = _build(n, e, d)

    src = edge_index[0]
    dst = edge_index[1]
    src_s = src.reshape(NS, (e // NS) // CH, CH)
    dst_s = dst.reshape(NS, (e // NS) // CH, CH)
    rpt, _ = _row_part(n)
    ones16 = jnp.ones((rpt,), jnp.float32)
    zeros16 = jnp.zeros((rpt,), jnp.float32)
    b1r, g1r, be1r = b1.reshape(1, d), g1.reshape(1, d), be1.reshape(1, d)
    b2r, g2r, be2r = b2.reshape(1, d), g2.reshape(1, d), be2.reshape(1, d)

    degp = deg_call(dst_s, ones16, zeros16)
    ys = tc1(x, W1, degp)
    acc1 = agg_call(ys, src_s, dst_s)
    ys2 = tc2(acc1, degp, b1r, g1r, be1r, W2)
    acc2 = agg_call(ys2, src_s, dst_s)
    out = tc3(acc2, degp, b2r, g2r, be2r, x)
    return out


# speedup vs baseline: 1.1691x; 1.0882x over previous
import functools

import jax
import jax.numpy as jnp
from jax import lax
from jax.experimental import pallas as pl
from jax.experimental.pallas import tpu as pltpu
from jax.experimental.pallas import tpu_sc as plsc

NC = 2
NS = 16
NW = NC * NS
CH = 80
NBUF = 6
LAG = 2

_MESH = plsc.VectorSubcoreMesh(core_axis_name="c", subcore_axis_name="s")
_SC_PARAMS = pltpu.CompilerParams(use_tc_tiling_on_sc=False)


def _row_part(n):
    rpt = ((n // NS + 7) // 8) * 8
    tail = n - rpt * (NS - 1)
    assert 0 < tail <= rpt and tail % 8 == 0
    return rpt, tail


def _per_tile_rows(s, n, copy_fn):
    rpt, tail = _row_part(n)

    @pl.when(s < NS - 1)
    def _():
        copy_fn(s * rpt, rpt)

    @pl.when(s == NS - 1)
    def _():
        copy_fn((NS - 1) * rpt, tail)


def _deg_body(nchunks, n, dst_hbm, ones_hbm, zeros_hbm, out_hbm,
              dst_v, ones_v, deg_sh, sem):
    c = lax.axis_index("c")
    s = lax.axis_index("s")

    def init(r0, nr):
        @pl.when(c == 0)
        def _():
            pltpu.sync_copy(ones_hbm.at[pl.ds(0, nr)],
                            deg_sh.at[pl.ds(r0, nr)])

        @pl.when(c == 1)
        def _():
            pltpu.sync_copy(zeros_hbm.at[pl.ds(0, nr)],
                            deg_sh.at[pl.ds(r0, nr)])

    _per_tile_rows(s, n, init)
    pltpu.sync_copy(ones_hbm.at[pl.ds(0, CH)], ones_v)
    pltpu.sync_copy(dst_hbm.at[s, pl.ds(c * nchunks, nchunks), :], dst_v)
    plsc.subcore_barrier()

    def step(j, carry):
        pltpu.async_copy(ones_v, deg_sh.at[dst_v.at[j]], sem, add=True)

        @pl.when(j >= 8)
        def _():
            pltpu.make_async_copy(ones_v, deg_sh.at[dst_v.at[j]], sem).wait()

        return carry

    lax.fori_loop(0, nchunks, step, 0)

    def drain(j, carry):
        pltpu.make_async_copy(ones_v, deg_sh.at[dst_v.at[0]], sem).wait()
        return carry

    lax.fori_loop(0, 8, drain, 0)
    plsc.subcore_barrier()

    def writeout(r0, nr):
        pltpu.sync_copy(deg_sh.at[pl.ds(r0, nr)],
                        out_hbm.at[c, pl.ds(r0, nr)])

    _per_tile_rows(s, n, writeout)


def _agg_body(nchunks, n, ys_hbm, src_hbm, dst_hbm, out_hbm,
              src_v, dst_v, rows_v, acc_sh, sem_g, sem_s):
    c = lax.axis_index("c")
    s = lax.axis_index("s")
    yc = ys_hbm.at[c]

    def init(r0, nr):
        pltpu.sync_copy(yc.at[pl.ds(r0, nr), :], acc_sh.at[pl.ds(r0, nr), :])

    _per_tile_rows(s, n, init)
    pltpu.sync_copy(src_hbm.at[s], src_v)
    pltpu.sync_copy(dst_hbm.at[s], dst_v)
    plsc.subcore_barrier()

    for b in range(NBUF - LAG):
        pltpu.async_copy(yc.at[src_v.at[b]], rows_v.at[b], sem_g)

    def step(j, carry):
        b = lax.rem(j, NBUF)
        pltpu.make_async_copy(yc.at[src_v.at[j]], rows_v.at[b], sem_g).wait()
        pltpu.async_copy(rows_v.at[b], acc_sh.at[dst_v.at[j]], sem_s,
                         add=True)

        @pl.when(j >= LAG)
        def _():
            pltpu.make_async_copy(rows_v.at[0], acc_sh.at[dst_v.at[0]],
                                  sem_s).wait()

        @pl.when(j + NBUF - LAG < nchunks)
        def _():
            pltpu.async_copy(yc.at[src_v.at[j + NBUF - LAG]],
                             rows_v.at[lax.rem(j + NBUF - LAG, NBUF)], sem_g)

        return carry

    lax.fori_loop(0, nchunks, step, 0)

    def drain(j, carry):
        pltpu.make_async_copy(rows_v.at[0], acc_sh.at[dst_v.at[0]],
                              sem_s).wait()
        return carry

    lax.fori_loop(0, LAG, drain, 0)
    plsc.subcore_barrier()

    dh = acc_sh.shape[1]

    def writeout(r0, nr):
        pltpu.sync_copy(acc_sh.at[pl.ds(r0, nr), :],
                        out_hbm.at[pl.ds(r0, nr), pl.ds(c * dh, dh)])

    _per_tile_rows(s, n, writeout)


def _dinv(degp_ref):
    p = degp_ref[0] + degp_ref[1]
    return lax.rsqrt(p)[:, None]


def _tc1_body(x_ref, w1_ref, degp_ref, ys_ref):
    y = jnp.dot(x_ref[...], w1_ref[...], preferred_element_type=jnp.float32)
    ys = y * _dinv(degp_ref)
    dh = ys.shape[1] // 2
    ys_ref[0] = ys[:, :dh]
    ys_ref[1] = ys[:, dh:]


def _bn(z, g, be):
    m = jnp.mean(z, axis=0, keepdims=True)
    zc = z - m
    v = jnp.mean(zc * zc, axis=0, keepdims=True)
    return zc * lax.rsqrt(v + 1e-5) * g + be


def _tc2_body(acc_ref, degp_ref, b1_ref, g1_ref, be1_ref, w2_ref, ys_ref):
    dinv = _dinv(degp_ref)
    z = acc_ref[...] * dinv + b1_ref[...]
    h = _bn(z, g1_ref[...], be1_ref[...])
    h = jnp.where(h > 0, h, 0.01 * h)
    y = jnp.dot(h, w2_ref[...], preferred_element_type=jnp.float32)
    ys = y * dinv
    dh = ys.shape[1] // 2
    ys_ref[0] = ys[:, :dh]
    ys_ref[1] = ys[:, dh:]


def _tc3_body(acc_ref, degp_ref, b2_ref, g2_ref, be2_ref, x_ref, out_ref):
    z = acc_ref[...] * _dinv(degp_ref) + b2_ref[...]
    h = _bn(z, g2_ref[...], be2_ref[...])
    t = h + x_ref[...]
    out_ref[...] = jnp.where(t > 0, t, 0.01 * t)


@functools.lru_cache(maxsize=None)
def _build(n, e, d):
    assert e % NW == 0 and (e // NS) % CH == 0 and n % 8 == 0 and d % 2 == 0
    epw = e // NW
    eps = e // NS
    dh = d // 2

    deg_call = pl.kernel(
        functools.partial(_deg_body, epw // CH, n),
        out_type=jax.ShapeDtypeStruct((NC, n), jnp.float32),
        mesh=_MESH,
        scratch_types=[
            pltpu.VMEM((epw // CH, CH), jnp.int32),
            pltpu.VMEM((CH,), jnp.float32),
            pltpu.VMEM_SHARED((n,), jnp.float32),
            pltpu.SemaphoreType.DMA,
        ],
        compiler_params=_SC_PARAMS,
    )

    agg_call = pl.kernel(
        functools.partial(_agg_body, eps // CH, n),
        out_type=jax.ShapeDtypeStruct((n, d), jnp.float32),
        mesh=_MESH,
        scratch_types=[
            pltpu.VMEM((eps // CH, CH), jnp.int32),
            pltpu.VMEM((eps // CH, CH), jnp.int32),
            pltpu.VMEM((NBUF, CH, dh), jnp.float32),
            pltpu.VMEM_SHARED((n, dh), jnp.float32),
            pltpu.SemaphoreType.DMA,
            pltpu.SemaphoreType.DMA,
        ],
        compiler_params=_SC_PARAMS,
    )

    tc1 = pl.pallas_call(
        _tc1_body,
        out_shape=jax.ShapeDtypeStruct((NC, n, dh), jnp.float32),
    )
    tc2 = pl.pallas_call(
        _tc2_body,
        out_shape=jax.ShapeDtypeStruct((NC, n, dh), jnp.float32),
    )
    tc3 = pl.pallas_call(
        _tc3_body,
        out_shape=jax.ShapeDtypeStruct((n, d), jnp.float32),
    )
    return deg_call, agg_call, tc1, tc2, tc3


def kernel(x, edge_index, W1, b1, g1, be1, W2, b2, g2, be2):
    n, d = x.shape
    e = edge_index.shape[1]
    deg_call, agg_call, tc1, tc2, tc3 = _build(n, e, d)

    src = edge_index[0]
    dst = edge_index[1]
    src_s = src.reshape(NS, (e // NS) // CH, CH)
    dst_s = dst.reshape(NS, (e // NS) // CH, CH)
    rpt, _ = _row_part(n)
    ones16 = jnp.ones((rpt,), jnp.float32)
    zeros16 = jnp.zeros((rpt,), jnp.float32)
    b1r, g1r, be1r = b1.reshape(1, d), g1.reshape(1, d), be1.reshape(1, d)
    b2r, g2r, be2r = b2.reshape(1, d), g2.reshape(1, d), be2.reshape(1, d)

    degp = deg_call(dst_s, ones16, zeros16)
    ys = tc1(x, W1, degp)
    acc1 = agg_call(ys, src_s, dst_s)
    ys2 = tc2(acc1, degp, b1r, g1r, be1r, W2)
    acc2 = agg_call(ys2, src_s, dst_s)
    out = tc3(acc2, degp, b2r, g2r, be2r, x)
    return out
